# Initial kernel scaffold; baseline (speedup 1.0000x reference)
import jax
import jax.numpy as jnp
from jax.experimental import pallas as pl


def kernel(weights, condition, codebook):
    raise NotImplementedError("write your pallas kernel here")


# speedup vs baseline: 1.1848x; 1.1848x over previous
import functools

import jax
import jax.numpy as jnp
from jax import lax
from jax.experimental import pallas as pl
from jax.experimental.pallas import tpu as pltpu
from jax.experimental.pallas import tpu_sc as plsc

N_TOKENS = 8192
CODE_DIM = 64
CODEBOOK_SIZE = 8192

_BN = 512


def _argmin_body(zsq_ref, z_ref, cb_ref, csq_ref, idx_ref):
    z = z_ref[...]
    cb = cb_ref[...]
    mt = lax.dot_general(cb.astype(jnp.bfloat16), z.astype(jnp.bfloat16),
                         (((1,), (1,)), ((), ())),
                         preferred_element_type=jnp.float32)
    d = (zsq_ref[...] - 2.0 * mt) + csq_ref[...]
    mins = jnp.min(d, axis=0, keepdims=True)
    iota = lax.broadcasted_iota(jnp.int32, d.shape, 0)
    masked = jnp.where(d == mins, iota, jnp.int32(2**31 - 1))
    idx_ref[...] = jnp.min(masked, axis=0, keepdims=True)


def _compute_indices(z, zsq_row, codebook, csq_col):
    grid = N_TOKENS // _BN
    out = pl.pallas_call(
        _argmin_body,
        grid=(grid,),
        in_specs=[
            pl.BlockSpec((1, _BN), lambda i: (0, i)),
            pl.BlockSpec((_BN, CODE_DIM), lambda i: (i, 0)),
            pl.BlockSpec((CODEBOOK_SIZE, CODE_DIM), lambda i: (0, 0)),
            pl.BlockSpec((CODEBOOK_SIZE, 1), lambda i: (0, 0)),
        ],
        out_specs=pl.BlockSpec((1, _BN), lambda i: (0, i)),
        out_shape=jax.ShapeDtypeStruct((1, N_TOKENS), jnp.int32),
        compiler_params=pltpu.CompilerParams(
            dimension_semantics=("arbitrary",),
        ),
    )(zsq_row, z, codebook, csq_col)
    return out[0]


_GD = 128
_GC = 128


def _sc_gather_build():
    info = plsc.get_sparse_core_info()
    nc, ns = info.num_cores, info.num_subcores
    nw = nc * ns
    b_per_w = N_TOKENS // nw
    n_chunks = b_per_w // _GC
    mesh = plsc.VectorSubcoreMesh(core_axis_name="c", subcore_axis_name="s")

    @functools.partial(
        pl.kernel,
        mesh=mesh,
        out_type=jax.ShapeDtypeStruct((N_TOKENS, _GD), jnp.float32),
        scratch_types=[
            pltpu.VMEM((n_chunks, _GC), jnp.int32),
            pltpu.VMEM((b_per_w, _GD), jnp.float32),
            pltpu.SemaphoreType.DMA,
        ],
    )
    def gather(idx_hbm, table_hbm, out_hbm, idx_v, rows_v, sem):
        wid = lax.axis_index("s") * nc + lax.axis_index("c")
        base = wid * b_per_w
        pltpu.sync_copy(idx_hbm.at[pl.ds(wid * n_chunks, n_chunks)], idx_v)
        copies = [
            pltpu.async_copy(table_hbm.at[idx_v.at[j]],
                             rows_v.at[pl.ds(j * _GC, _GC)], sem)
            for j in range(n_chunks)
        ]
        for cp in copies:
            cp.wait()
        pltpu.sync_copy(rows_v, out_hbm.at[pl.ds(base, b_per_w)])

    return gather


def kernel(weights, condition, codebook):
    z = weights - condition
    zsq = jnp.sum(z * z, axis=1, keepdims=True)
    csq = jnp.sum(codebook * codebook, axis=1)[None, :]
    indices = _compute_indices(z, zsq.T, codebook, csq.T)
    cb_pad = jnp.pad(codebook, ((0, 0), (0, _GD - CODE_DIM)))
    idx2d = indices.reshape(N_TOKENS // _GC, _GC)
    out_pad = _sc_gather_build()(idx2d, cb_pad)
    return (indices, out_pad[:, :CODE_DIM])
